# baseline (speedup 1.0000x reference)
import functools

import jax
import jax.numpy as jnp
from jax import lax
from jax.experimental import pallas as pl
from jax.experimental.pallas import tpu as pltpu
from jax.experimental.pallas import tpu_sc as plsc

N = 10000
K = 16
C = 128
NP = 10112
EPS = 1e-5

RB = 128
GRID = NP // RB

NW = 32
PER_W = NP * K // NW
CH = 64
NCH = PER_W // CH

FAR = 1.0e6


def _prep_body(f_ref, x16_ref, wq_ref, wk_ref, wv_ref, bq_ref, bk_ref, bv_ref,
               wp1_ref, bp1_ref, psc_ref, psh_ref, wp2_ref, bp2_ref,
               q_ref, kv_ref):
    f = f_ref[...]
    q = jnp.dot(f, wq_ref[...], preferred_element_type=jnp.float32) + bq_ref[...]
    k = jnp.dot(f, wk_ref[...], preferred_element_type=jnp.float32) + bk_ref[...]
    v = jnp.dot(f, wv_ref[...], preferred_element_type=jnp.float32) + bv_ref[...]
    x16 = x16_ref[...]
    p1 = jnp.dot(x16, wp1_ref[...], preferred_element_type=jnp.float32) + bp1_ref[...]
    p1 = jnp.maximum(p1 * psc_ref[...] + psh_ref[...], 0.0)
    p2 = jnp.dot(p1, wp2_ref[...], preferred_element_type=jnp.float32) + bp2_ref[...]
    q_ref[...] = q
    kv_ref[...] = jnp.concatenate([k + p2, v + p2], axis=1)


NPC = 10240
NSC = 40
NSCP = 48
NRW = NP // NW
FBIG = 3.0e38
IBIG = 2 ** 30


def _knn_sc(xc, yc, zc):
    mesh = plsc.VectorSubcoreMesh(core_axis_name="c", subcore_axis_name="s")

    @functools.partial(
        pl.kernel, mesh=mesh,
        out_type=jax.ShapeDtypeStruct((NP * K,), jnp.int32),
        scratch_types=[pltpu.VMEM((NPC,), jnp.float32),
                       pltpu.VMEM((NPC,), jnp.float32),
                       pltpu.VMEM((NPC,), jnp.float32),
                       pltpu.VMEM((NPC,), jnp.float32),
                       pltpu.VMEM((16, NPC // 16), jnp.float32),
                       pltpu.VMEM((16, NSCP), jnp.float32),
                       pltpu.VMEM((16,), jnp.int32)],
        compiler_params=pltpu.CompilerParams(needs_layout_passes=False),
    )
    def knn_kernel(xc_hbm, yc_hbm, zc_hbm, idx_out, xv, yv, zv, sqv, d2t, smin,
                   outv):
        iota = lax.iota(jnp.int32, 16)
        wid = lax.axis_index("s") * 2 + lax.axis_index("c")
        pltpu.sync_copy(xc_hbm, xv)
        pltpu.sync_copy(yc_hbm, yv)
        pltpu.sync_copy(zc_hbm, zv)
        inf16 = jnp.full((16,), FBIG, jnp.float32)
        for r16 in range(16):
            for kk in range(NSCP // 16):
                smin[r16, pl.ds(kk * 16, 16)] = inf16

        def bf16_round(v):
            b = plsc.bitcast(v, jnp.int32)
            r = (b + 0x7FFF + ((b >> 16) & 1)) & ~0xFFFF
            return plsc.bitcast(r, jnp.float32)

        def pre_body(c, carry):
            off = pl.multiple_of(c * 16, 16)
            x = xv[pl.ds(off, 16)]
            y = yv[pl.ds(off, 16)]
            z = zv[pl.ds(off, 16)]
            sqv[pl.ds(off, 16)] = x * x + y * y + z * z
            xv[pl.ds(off, 16)] = bf16_round(x)
            yv[pl.ds(off, 16)] = bf16_round(y)
            zv[pl.ds(off, 16)] = bf16_round(z)
            return carry

        lax.fori_loop(0, NPC // 16, pre_body, 0)

        def lane_bcast(vec_ref, i):
            base = pl.multiple_of((i // 16) * 16, 16)
            chunk = vec_ref[pl.ds(base, 16)]
            s = jnp.min(jnp.where(iota == i % 16, chunk, FBIG))
            return s, jnp.full((16,), s, jnp.float32)

        def row_body(r, carry):
            i = wid * NRW + r
            _, xi = lane_bcast(xv, i)
            _, yi = lane_bcast(yv, i)
            _, zi = lane_bcast(zv, i)
            _, sqi = lane_bcast(sqv, i)

            def sc_body(sc, lmin):
                sacc = inf16
                for u in range(16):
                    c = sc * 16 + u
                    off = pl.multiple_of(c * 16, 16)
                    t = xv[pl.ds(off, 16)] * xi
                    t = t + yv[pl.ds(off, 16)] * yi
                    t = t + zv[pl.ds(off, 16)] * zi
                    d2 = (sqi + sqv[pl.ds(off, 16)]) - 2.0 * t
                    plsc.store_scatter(d2t, [iota, jnp.full((16,), c, jnp.int32)], d2)
                    sacc = jnp.minimum(sacc, d2)
                plsc.store_scatter(smin, [iota, jnp.full((16,), sc, jnp.int32)], sacc)
                return jnp.minimum(lmin, sacc)

            lmin = lax.fori_loop(0, NSC, sc_body, inf16)

            def ext_body(t, carry):
                lmin, outacc = carry
                m = jnp.min(lmin)
                mv = jnp.full((16,), m, jnp.float32)
                lstar = jnp.min(jnp.where(lmin == mv, iota, IBIG))
                sstar = IBIG
                for kk in range(NSCP // 16):
                    srow = smin[lstar, pl.ds(kk * 16, 16)]
                    sstar = jnp.minimum(
                        sstar, jnp.min(jnp.where(srow == mv, iota + kk * 16, IBIG)))
                off = pl.multiple_of(sstar * 16, 16)
                dchunk = d2t[lstar, pl.ds(off, 16)]
                pstar = jnp.min(jnp.where(dchunk == mv, iota + off, IBIG))
                jstar = pstar * 16 + lstar
                outacc = jnp.where(iota == t,
                                   jnp.full((16,), jstar, jnp.int32), outacc)
                lst16 = jnp.full((16,), lstar, jnp.int32)
                plsc.store_scatter(d2t, [lst16, jnp.full((16,), pstar, jnp.int32)],
                                   inf16, mask=iota == 0)
                cmin = jnp.min(d2t[lstar, pl.ds(off, 16)])
                plsc.store_scatter(smin, [lst16, jnp.full((16,), sstar, jnp.int32)],
                                   jnp.full((16,), cmin, jnp.float32), mask=iota == 0)
                lnew = FBIG
                for kk in range(NSCP // 16):
                    lnew = jnp.minimum(lnew, jnp.min(smin[lstar, pl.ds(kk * 16, 16)]))
                lmin = jnp.where(iota == lstar, jnp.full((16,), lnew, jnp.float32), lmin)
                return lmin, outacc

            _, outacc = lax.fori_loop(0, K, ext_body,
                                      (lmin, jnp.zeros((16,), jnp.int32)))
            outv[...] = jnp.minimum(outacc, NP - 1)
            pltpu.sync_copy(outv, idx_out.at[pl.ds(i * K, K)])
            return carry

        lax.fori_loop(0, NRW, row_body, 0)

    return knn_kernel(xc, yc, zc)


def _attn_body(kvg_ref, q_ref, w1sc_ref, w1sh_ref, ww1_ref, bw1_ref,
               w2sc_ref, w2sh_ref, ww2_ref, bw2_ref, t16_ref, out_ref):
    kv = kvg_ref[...]
    kp = kv[:, :C]
    vp = kv[:, C:]
    q = q_ref[...]

    qb = jnp.broadcast_to(q[:, None, :], (RB, K, C)).reshape(RB * K, C)
    w = kp - qb
    w = jnp.maximum(w * w1sc_ref[...] + w1sh_ref[...], 0.0)
    w = jnp.dot(w, ww1_ref[...], preferred_element_type=jnp.float32) + bw1_ref[...]
    w = jnp.maximum(w * w2sc_ref[...] + w2sh_ref[...], 0.0)
    w = jnp.dot(w, ww2_ref[...], preferred_element_type=jnp.float32) + bw2_ref[...]

    w3 = w.reshape(RB, K, 16)
    m = jnp.max(w3, axis=1, keepdims=True)
    e = jnp.exp(w3 - m)
    sm = (e / jnp.sum(e, axis=1, keepdims=True)).reshape(RB * K, 16)
    wfull = jnp.dot(sm, t16_ref[...], preferred_element_type=jnp.float32)

    out_ref[...] = jnp.sum((vp * wfull).reshape(RB, K, C), axis=1)


def _sc_gather(kv_t, idxw):
    mesh = plsc.VectorSubcoreMesh(core_axis_name="c", subcore_axis_name="s")

    @functools.partial(
        pl.kernel, mesh=mesh,
        out_type=jax.ShapeDtypeStruct((NP * K, 2 * C), jnp.float32),
        scratch_types=[pltpu.VMEM((NCH, CH), jnp.int32),
                       pltpu.VMEM((CH, 2 * C), jnp.float32),
                       pltpu.SemaphoreType.DMA],
    )
    def gather_kernel(kv_hbm, idx_hbm, kvg_out, idx_v, rows_kv, sem):
        wid = lax.axis_index("s") * 2 + lax.axis_index("c")
        pltpu.sync_copy(idx_hbm.at[wid], idx_v)
        base = wid * PER_W

        def body(c, carry):
            pltpu.async_copy(kv_hbm.at[idx_v.at[c]], rows_kv, sem).wait()
            pltpu.sync_copy(rows_kv, kvg_out.at[pl.ds(base + c * CH, CH)])
            return carry

        lax.fori_loop(0, NCH, body, 0)

    return gather_kernel(kv_t, idxw)


def kernel(xyz, features, Wq, bq, Wk, bk, Wv, bv, Wp1, bp1, p_g, p_b, p_m, p_v,
           Wp2, bp2, w_g1, w_b1, w_m1, w_v1, Ww1, bw1, w_g2, w_b2, w_m2, w_v2,
           Ww2, bw2):
    f32 = jnp.float32

    xyz0 = xyz[0]
    xyz8 = jnp.zeros((NP, 8), f32)
    xyz8 = xyz8.at[:N, :3].set(xyz0)
    xyz8 = xyz8.at[N:, 0].set(FAR)
    x16 = jnp.concatenate([xyz8, jnp.zeros((NP, 8), f32)], axis=1)
    xc = jnp.full((NPC,), FAR, f32).at[:N].set(xyz0[:, 0])
    yc = jnp.zeros((NPC,), f32).at[:N].set(xyz0[:, 1])
    zc = jnp.zeros((NPC,), f32).at[:N].set(xyz0[:, 2])

    featT = jnp.pad(features[0].T, ((0, NP - N), (0, 0)))

    def bn_fold(g, b, m, v):
        sc = g / jnp.sqrt(v + EPS)
        return sc, b - m * sc

    psc, psh = bn_fold(p_g, p_b, p_m, p_v)
    w1sc, w1sh = bn_fold(w_g1, w_b1, w_m1, w_v1)
    w2sc, w2sh = bn_fold(w_g2, w_b2, w_m2, w_v2)

    wp1p = jnp.zeros((16, 16), f32).at[:3, :3].set(Wp1.T)
    bp1p = jnp.zeros((1, 16), f32).at[0, :3].set(bp1)
    pscp = jnp.ones((1, 16), f32).at[0, :3].set(psc)
    pshp = jnp.zeros((1, 16), f32).at[0, :3].set(psh)
    wp2p = jnp.zeros((16, C), f32).at[:3, :].set(Wp2.T)

    t16 = (lax.broadcasted_iota(jnp.int32, (16, C), 1) % 16 ==
           lax.broadcasted_iota(jnp.int32, (16, C), 0)).astype(f32)

    q_t, kv_t = pl.pallas_call(
        _prep_body,
        grid=(GRID,),
        in_specs=[pl.BlockSpec((RB, C), lambda i: (i, 0)),
                  pl.BlockSpec((RB, 16), lambda i: (i, 0))] +
                 [pl.BlockSpec((C, C), lambda i: (0, 0))] * 3 +
                 [pl.BlockSpec((1, C), lambda i: (0, 0))] * 3 +
                 [pl.BlockSpec((16, 16), lambda i: (0, 0)),
                  pl.BlockSpec((1, 16), lambda i: (0, 0)),
                  pl.BlockSpec((1, 16), lambda i: (0, 0)),
                  pl.BlockSpec((1, 16), lambda i: (0, 0)),
                  pl.BlockSpec((16, C), lambda i: (0, 0)),
                  pl.BlockSpec((1, C), lambda i: (0, 0))],
        out_specs=[pl.BlockSpec((RB, C), lambda i: (i, 0)),
                   pl.BlockSpec((RB, 2 * C), lambda i: (i, 0))],
        out_shape=[jax.ShapeDtypeStruct((NP, C), f32),
                   jax.ShapeDtypeStruct((NP, 2 * C), f32)],
    )(featT, x16, Wq.T, Wk.T, Wv.T, bq[None], bk[None], bv[None],
      wp1p, bp1p, pscp, pshp, wp2p, bp2[None])

    idx = _knn_sc(xc, yc, zc).reshape(NP, K)

    idxw = idx.reshape(NW, NCH, CH)
    kvg = _sc_gather(kv_t, idxw)

    wspec = lambda shape: pl.BlockSpec(shape, lambda i: (0, 0))
    out = pl.pallas_call(
        _attn_body,
        grid=(GRID,),
        in_specs=[pl.BlockSpec((RB * K, 2 * C), lambda i: (i, 0)),
                  pl.BlockSpec((RB, C), lambda i: (i, 0)),
                  wspec((1, C)), wspec((1, C)),
                  wspec((C, 16)), wspec((1, 16)), wspec((1, 16)), wspec((1, 16)),
                  wspec((16, 16)), wspec((1, 16)), wspec((16, C))],
        out_specs=pl.BlockSpec((RB, C), lambda i: (i, 0)),
        out_shape=jax.ShapeDtypeStruct((NP, C), f32),
    )(kvg, q_t,
      w1sc[None], w1sh[None],
      Ww1.T, bw1[None], w2sc[None], w2sh[None],
      Ww2.T, bw2[None], t16)

    return out[:N].T[None]


# speedup vs baseline: 2.8608x; 2.8608x over previous
import functools

import jax
import jax.numpy as jnp
from jax import lax
from jax.experimental import pallas as pl
from jax.experimental.pallas import tpu as pltpu
from jax.experimental.pallas import tpu_sc as plsc

N = 10000
K = 16
C = 128
NP = 10112
EPS = 1e-5

RB = 128
GRID = NP // RB

NW = 32
PER_W = NP * K // NW
CH = 64
NCH = PER_W // CH

FAR = 1.0e6


def _prep_body(f_ref, x16_ref, wq_ref, wk_ref, wv_ref, bq_ref, bk_ref, bv_ref,
               wp1_ref, bp1_ref, psc_ref, psh_ref, wp2_ref, bp2_ref,
               q_ref, kv_ref):
    f = f_ref[...]
    q = jnp.dot(f, wq_ref[...], preferred_element_type=jnp.float32) + bq_ref[...]
    k = jnp.dot(f, wk_ref[...], preferred_element_type=jnp.float32) + bk_ref[...]
    v = jnp.dot(f, wv_ref[...], preferred_element_type=jnp.float32) + bv_ref[...]
    x16 = x16_ref[...]
    p1 = jnp.dot(x16, wp1_ref[...], preferred_element_type=jnp.float32) + bp1_ref[...]
    p1 = jnp.maximum(p1 * psc_ref[...] + psh_ref[...], 0.0)
    p2 = jnp.dot(p1, wp2_ref[...], preferred_element_type=jnp.float32) + bp2_ref[...]
    q_ref[...] = q
    kv_ref[...] = jnp.concatenate([k + p2, v + p2], axis=1)


NPC = 10240
NSC = 40
NSCP = 48
NRW = NP // NW
FBIG = 3.0e38
IBIG = 2 ** 30


def _knn_sc(xc, yc, zc):
    mesh = plsc.VectorSubcoreMesh(core_axis_name="c", subcore_axis_name="s")

    @functools.partial(
        pl.kernel, mesh=mesh,
        out_type=jax.ShapeDtypeStruct((NP * K,), jnp.int32),
        scratch_types=[pltpu.VMEM((NPC,), jnp.float32),
                       pltpu.VMEM((NPC,), jnp.float32),
                       pltpu.VMEM((NPC,), jnp.float32),
                       pltpu.VMEM((NPC,), jnp.float32),
                       pltpu.VMEM((NSCP * 16,), jnp.float32),
                       pltpu.VMEM((4 * K,), jnp.int32)],
        compiler_params=pltpu.CompilerParams(needs_layout_passes=False),
    )
    def knn_kernel(xc_hbm, yc_hbm, zc_hbm, idx_out, xv, yv, zv, sqv, smin2,
                   outv):
        iota = lax.iota(jnp.int32, 16)
        wid = lax.axis_index("s") * 2 + lax.axis_index("c")
        pltpu.sync_copy(xc_hbm, xv)
        pltpu.sync_copy(yc_hbm, yv)
        pltpu.sync_copy(zc_hbm, zv)
        inf16 = jnp.full((16,), FBIG, jnp.float32)
        for pads in range(NSC * 16, NSCP * 16, 16):
            smin2[pl.ds(pads, 16)] = inf16

        def bf16_round(v):
            b = plsc.bitcast(v, jnp.int32)
            r = (b + 0x7FFF + ((b >> 16) & 1)) & ~0xFFFF
            return plsc.bitcast(r, jnp.float32)

        def pre_body(c, carry):
            off = pl.multiple_of(c * 16, 16)
            x = xv[pl.ds(off, 16)]
            y = yv[pl.ds(off, 16)]
            z = zv[pl.ds(off, 16)]
            sqv[pl.ds(off, 16)] = x * x + y * y + z * z
            xv[pl.ds(off, 16)] = bf16_round(x)
            yv[pl.ds(off, 16)] = bf16_round(y)
            zv[pl.ds(off, 16)] = bf16_round(z)
            return carry

        lax.fori_loop(0, NPC // 16, pre_body, 0)

        def lane_bcast(vec_ref, i):
            base = pl.multiple_of((i // 16) * 16, 16)
            chunk = vec_ref[pl.ds(base, 16)]
            s = jnp.min(jnp.where(iota == i % 16, chunk, FBIG))
            return jnp.full((16,), s, jnp.float32)

        def row_body(g, carry):
            for r4 in range(4):
                r = g * 4 + r4
                i = wid * NRW + r
                xi = lane_bcast(xv, i)
                yi = lane_bcast(yv, i)
                zi = lane_bcast(zv, i)
                sqi = lane_bcast(sqv, i)

                def dist16(off):
                    t = xv[pl.ds(off, 16)] * xi
                    t = t + yv[pl.ds(off, 16)] * yi
                    t = t + zv[pl.ds(off, 16)] * zi
                    return (sqi + sqv[pl.ds(off, 16)]) - 2.0 * t

                def dist_gather(jv):
                    t = plsc.load_gather(xv, [jv]) * xi
                    t = t + plsc.load_gather(yv, [jv]) * yi
                    t = t + plsc.load_gather(zv, [jv]) * zi
                    return (sqi + plsc.load_gather(sqv, [jv])) - 2.0 * t

                def sc_body(sc, lmin):
                    sacc = inf16
                    for u in range(16):
                        off = pl.multiple_of((sc * 16 + u) * 16, 16)
                        sacc = jnp.minimum(sacc, dist16(off))
                    smin2[pl.ds(pl.multiple_of(sc * 16, 16), 16)] = sacc
                    return jnp.minimum(lmin, sacc)

                lmin = lax.fori_loop(0, NSC, sc_body, inf16)

                def ext_body(t, carry):
                    lmin, outacc = carry
                    m = jnp.min(lmin)
                    mv = jnp.full((16,), m, jnp.float32)
                    lstar = jnp.min(jnp.where(lmin == mv, iota, IBIG))
                    sstar = IBIG
                    srows = []
                    for kk in range(NSCP // 16):
                        sidx = (kk * 16 + iota) * 16 + lstar
                        srow = plsc.load_gather(smin2, [sidx])
                        srows.append(srow)
                        sstar = jnp.minimum(
                            sstar,
                            jnp.min(jnp.where(srow == mv, iota + kk * 16, IBIG)))
                    jv = (sstar * 16 + iota) * 16 + lstar
                    d2v = dist_gather(jv)
                    cstar = jnp.min(jnp.where(d2v == mv, sstar * 16 + iota, IBIG))
                    jstar = cstar * 16 + lstar
                    outacc = jnp.where(iota == t,
                                       jnp.full((16,), jstar, jnp.int32), outacc)
                    cmin = jnp.min(jnp.where(d2v <= mv, FBIG, d2v))
                    cminv = jnp.full((16,), cmin, jnp.float32)
                    plsc.store_scatter(
                        smin2, [jnp.full((16,), sstar * 16 + lstar, jnp.int32)],
                        cminv, mask=iota == 0)
                    lnew = FBIG
                    sstarv = jnp.full((16,), sstar, jnp.int32)
                    for kk in range(NSCP // 16):
                        srow = jnp.where(kk * 16 + iota == sstarv, cminv,
                                         srows[kk])
                        lnew = jnp.minimum(lnew, jnp.min(srow))
                    lmin = jnp.where(iota == lstar,
                                     jnp.full((16,), lnew, jnp.float32), lmin)
                    return lmin, outacc

                _, outacc = lax.fori_loop(0, K, ext_body,
                                          (lmin, jnp.zeros((16,), jnp.int32)))
                outv[pl.ds(r4 * K, K)] = jnp.minimum(outacc, NP - 1)
            base_i = wid * NRW + g * 4
            pltpu.sync_copy(outv, idx_out.at[pl.ds(base_i * K, 4 * K)])
            return carry

        lax.fori_loop(0, NRW // 4, row_body, 0)

    return knn_kernel(xc, yc, zc)


def _attn_body(kvg_ref, q_ref, w1sc_ref, w1sh_ref, ww1_ref, bw1_ref,
               w2sc_ref, w2sh_ref, ww2_ref, bw2_ref, t16_ref, out_ref):
    kv = kvg_ref[...]
    kp = kv[:, :C]
    vp = kv[:, C:]
    q = q_ref[...]

    qb = jnp.broadcast_to(q[:, None, :], (RB, K, C)).reshape(RB * K, C)
    w = kp - qb
    w = jnp.maximum(w * w1sc_ref[...] + w1sh_ref[...], 0.0)
    w = jnp.dot(w, ww1_ref[...], preferred_element_type=jnp.float32) + bw1_ref[...]
    w = jnp.maximum(w * w2sc_ref[...] + w2sh_ref[...], 0.0)
    w = jnp.dot(w, ww2_ref[...], preferred_element_type=jnp.float32) + bw2_ref[...]

    w3 = w.reshape(RB, K, 16)
    m = jnp.max(w3, axis=1, keepdims=True)
    e = jnp.exp(w3 - m)
    sm = (e / jnp.sum(e, axis=1, keepdims=True)).reshape(RB * K, 16)
    wfull = jnp.dot(sm, t16_ref[...], preferred_element_type=jnp.float32)

    out_ref[...] = jnp.sum((vp * wfull).reshape(RB, K, C), axis=1)


def _sc_gather(kv_t, idxw):
    mesh = plsc.VectorSubcoreMesh(core_axis_name="c", subcore_axis_name="s")

    @functools.partial(
        pl.kernel, mesh=mesh,
        out_type=jax.ShapeDtypeStruct((NP * K, 2 * C), jnp.float32),
        scratch_types=[pltpu.VMEM((NCH, CH), jnp.int32),
                       pltpu.VMEM((CH, 2 * C), jnp.float32),
                       pltpu.SemaphoreType.DMA],
    )
    def gather_kernel(kv_hbm, idx_hbm, kvg_out, idx_v, rows_kv, sem):
        wid = lax.axis_index("s") * 2 + lax.axis_index("c")
        pltpu.sync_copy(idx_hbm.at[wid], idx_v)
        base = wid * PER_W

        def body(c, carry):
            pltpu.async_copy(kv_hbm.at[idx_v.at[c]], rows_kv, sem).wait()
            pltpu.sync_copy(rows_kv, kvg_out.at[pl.ds(base + c * CH, CH)])
            return carry

        lax.fori_loop(0, NCH, body, 0)

    return gather_kernel(kv_t, idxw)


def kernel(xyz, features, Wq, bq, Wk, bk, Wv, bv, Wp1, bp1, p_g, p_b, p_m, p_v,
           Wp2, bp2, w_g1, w_b1, w_m1, w_v1, Ww1, bw1, w_g2, w_b2, w_m2, w_v2,
           Ww2, bw2):
    f32 = jnp.float32

    xyz0 = xyz[0]
    xyz8 = jnp.zeros((NP, 8), f32)
    xyz8 = xyz8.at[:N, :3].set(xyz0)
    xyz8 = xyz8.at[N:, 0].set(FAR)
    x16 = jnp.concatenate([xyz8, jnp.zeros((NP, 8), f32)], axis=1)
    xc = jnp.full((NPC,), FAR, f32).at[:N].set(xyz0[:, 0])
    yc = jnp.zeros((NPC,), f32).at[:N].set(xyz0[:, 1])
    zc = jnp.zeros((NPC,), f32).at[:N].set(xyz0[:, 2])

    featT = jnp.pad(features[0].T, ((0, NP - N), (0, 0)))

    def bn_fold(g, b, m, v):
        sc = g / jnp.sqrt(v + EPS)
        return sc, b - m * sc

    psc, psh = bn_fold(p_g, p_b, p_m, p_v)
    w1sc, w1sh = bn_fold(w_g1, w_b1, w_m1, w_v1)
    w2sc, w2sh = bn_fold(w_g2, w_b2, w_m2, w_v2)

    wp1p = jnp.zeros((16, 16), f32).at[:3, :3].set(Wp1.T)
    bp1p = jnp.zeros((1, 16), f32).at[0, :3].set(bp1)
    pscp = jnp.ones((1, 16), f32).at[0, :3].set(psc)
    pshp = jnp.zeros((1, 16), f32).at[0, :3].set(psh)
    wp2p = jnp.zeros((16, C), f32).at[:3, :].set(Wp2.T)

    t16 = (lax.broadcasted_iota(jnp.int32, (16, C), 1) % 16 ==
           lax.broadcasted_iota(jnp.int32, (16, C), 0)).astype(f32)

    q_t, kv_t = pl.pallas_call(
        _prep_body,
        grid=(GRID,),
        in_specs=[pl.BlockSpec((RB, C), lambda i: (i, 0)),
                  pl.BlockSpec((RB, 16), lambda i: (i, 0))] +
                 [pl.BlockSpec((C, C), lambda i: (0, 0))] * 3 +
                 [pl.BlockSpec((1, C), lambda i: (0, 0))] * 3 +
                 [pl.BlockSpec((16, 16), lambda i: (0, 0)),
                  pl.BlockSpec((1, 16), lambda i: (0, 0)),
                  pl.BlockSpec((1, 16), lambda i: (0, 0)),
                  pl.BlockSpec((1, 16), lambda i: (0, 0)),
                  pl.BlockSpec((16, C), lambda i: (0, 0)),
                  pl.BlockSpec((1, C), lambda i: (0, 0))],
        out_specs=[pl.BlockSpec((RB, C), lambda i: (i, 0)),
                   pl.BlockSpec((RB, 2 * C), lambda i: (i, 0))],
        out_shape=[jax.ShapeDtypeStruct((NP, C), f32),
                   jax.ShapeDtypeStruct((NP, 2 * C), f32)],
    )(featT, x16, Wq.T, Wk.T, Wv.T, bq[None], bk[None], bv[None],
      wp1p, bp1p, pscp, pshp, wp2p, bp2[None])

    idx = _knn_sc(xc, yc, zc).reshape(NP, K)

    idxw = idx.reshape(NW, NCH, CH)
    kvg = _sc_gather(kv_t, idxw)

    wspec = lambda shape: pl.BlockSpec(shape, lambda i: (0, 0))
    out = pl.pallas_call(
        _attn_body,
        grid=(GRID,),
        in_specs=[pl.BlockSpec((RB * K, 2 * C), lambda i: (i, 0)),
                  pl.BlockSpec((RB, C), lambda i: (i, 0)),
                  wspec((1, C)), wspec((1, C)),
                  wspec((C, 16)), wspec((1, 16)), wspec((1, 16)), wspec((1, 16)),
                  wspec((16, 16)), wspec((1, 16)), wspec((16, C))],
        out_specs=pl.BlockSpec((RB, C), lambda i: (i, 0)),
        out_shape=jax.ShapeDtypeStruct((NP, C), f32),
    )(kvg, q_t,
      w1sc[None], w1sh[None],
      Ww1.T, bw1[None], w2sc[None], w2sh[None],
      Ww2.T, bw2[None], t16)

    return out[:N].T[None]


# speedup vs baseline: 3.9726x; 1.3887x over previous
import functools

import jax
import jax.numpy as jnp
from jax import lax
from jax.experimental import pallas as pl
from jax.experimental.pallas import tpu as pltpu
from jax.experimental.pallas import tpu_sc as plsc

N = 10000
K = 16
C = 128
NP = 10112
EPS = 1e-5

RB = 128
GRID = NP // RB

NW = 32
PER_W = NP * K // NW
CH = 64
NCH = PER_W // CH

FAR = 1.0e6


def _prep_body(f_ref, x16_ref, wq_ref, wk_ref, wv_ref, bq_ref, bk_ref, bv_ref,
               wp1_ref, bp1_ref, psc_ref, psh_ref, wp2_ref, bp2_ref,
               q_ref, kv_ref):
    f = f_ref[...]
    q = jnp.dot(f, wq_ref[...], preferred_element_type=jnp.float32) + bq_ref[...]
    k = jnp.dot(f, wk_ref[...], preferred_element_type=jnp.float32) + bk_ref[...]
    v = jnp.dot(f, wv_ref[...], preferred_element_type=jnp.float32) + bv_ref[...]
    x16 = x16_ref[...]
    p1 = jnp.dot(x16, wp1_ref[...], preferred_element_type=jnp.float32) + bp1_ref[...]
    p1 = jnp.maximum(p1 * psc_ref[...] + psh_ref[...], 0.0)
    p2 = jnp.dot(p1, wp2_ref[...], preferred_element_type=jnp.float32) + bp2_ref[...]
    q_ref[...] = q
    kv_ref[...] = jnp.concatenate([k + p2, v + p2], axis=1)


NPC = 10240
NSC = 40
NSCP = 48
NRW = NP // NW
FBIG = 3.0e38
IBIG = 2 ** 30


def _knn_sc(xc, yc, zc):
    mesh = plsc.VectorSubcoreMesh(core_axis_name="c", subcore_axis_name="s")

    @functools.partial(
        pl.kernel, mesh=mesh,
        out_type=jax.ShapeDtypeStruct((NP * K,), jnp.int32),
        scratch_types=[pltpu.VMEM((NPC,), jnp.float32),
                       pltpu.VMEM((NPC,), jnp.float32),
                       pltpu.VMEM((NPC,), jnp.float32),
                       pltpu.VMEM((NPC,), jnp.float32),
                       pltpu.VMEM((NSCP * 16,), jnp.float32),
                       pltpu.VMEM((NSCP * 16,), jnp.float32),
                       pltpu.VMEM((4 * K,), jnp.int32)],
        compiler_params=pltpu.CompilerParams(needs_layout_passes=False),
    )
    def knn_kernel(xc_hbm, yc_hbm, zc_hbm, idx_out, xv, yv, zv, sqv, smin2a,
                   smin2b, outv):
        iota = lax.iota(jnp.int32, 16)
        wid = lax.axis_index("s") * 2 + lax.axis_index("c")
        pltpu.sync_copy(xc_hbm, xv)
        pltpu.sync_copy(yc_hbm, yv)
        pltpu.sync_copy(zc_hbm, zv)
        inf16 = jnp.full((16,), FBIG, jnp.float32)
        for pads in range(NSC * 16, NSCP * 16, 16):
            smin2a[pl.ds(pads, 16)] = inf16
            smin2b[pl.ds(pads, 16)] = inf16

        def bf16_round(v):
            b = plsc.bitcast(v, jnp.int32)
            r = (b + 0x7FFF + ((b >> 16) & 1)) & ~0xFFFF
            return plsc.bitcast(r, jnp.float32)

        def pre_body(c, carry):
            off = pl.multiple_of(c * 16, 16)
            x = xv[pl.ds(off, 16)]
            y = yv[pl.ds(off, 16)]
            z = zv[pl.ds(off, 16)]
            sqv[pl.ds(off, 16)] = x * x + y * y + z * z
            xv[pl.ds(off, 16)] = bf16_round(x)
            yv[pl.ds(off, 16)] = bf16_round(y)
            zv[pl.ds(off, 16)] = bf16_round(z)
            return carry

        lax.fori_loop(0, NPC // 16, pre_body, 0)

        def lane_bcast(vec_ref, i):
            base = pl.multiple_of((i // 16) * 16, 16)
            chunk = vec_ref[pl.ds(base, 16)]
            s = jnp.min(jnp.where(iota == i % 16, chunk, FBIG))
            return jnp.full((16,), s, jnp.float32)

        def ext_one(t, lmin, outacc, smin2, xi, yi, zi, sqi):
            m = jnp.min(lmin)
            mv = jnp.full((16,), m, jnp.float32)
            lstar = jnp.min(jnp.where(lmin == mv, iota, IBIG))
            sstar = IBIG
            srows = []
            for kk in range(NSCP // 16):
                sidx = (kk * 16 + iota) * 16 + lstar
                srow = plsc.load_gather(smin2, [sidx])
                srows.append(srow)
                sstar = jnp.minimum(
                    sstar,
                    jnp.min(jnp.where(srow == mv, iota + kk * 16, IBIG)))
            jv = (sstar * 16 + iota) * 16 + lstar
            t1 = plsc.load_gather(xv, [jv]) * xi
            t1 = t1 + plsc.load_gather(yv, [jv]) * yi
            t1 = t1 + plsc.load_gather(zv, [jv]) * zi
            d2v = (sqi + plsc.load_gather(sqv, [jv])) - 2.0 * t1
            cstar = jnp.min(jnp.where(d2v == mv, sstar * 16 + iota, IBIG))
            jstar = cstar * 16 + lstar
            outacc = jnp.where(iota == t,
                               jnp.full((16,), jstar, jnp.int32), outacc)
            cmin = jnp.min(jnp.where(d2v <= mv, FBIG, d2v))
            cminv = jnp.full((16,), cmin, jnp.float32)
            plsc.store_scatter(
                smin2, [jnp.full((16,), sstar * 16 + lstar, jnp.int32)],
                cminv, mask=iota == 0)
            lnew = FBIG
            sstarv = jnp.full((16,), sstar, jnp.int32)
            for kk in range(NSCP // 16):
                srow = jnp.where(kk * 16 + iota == sstarv, cminv, srows[kk])
                lnew = jnp.minimum(lnew, jnp.min(srow))
            lmin = jnp.where(iota == lstar,
                             jnp.full((16,), lnew, jnp.float32), lmin)
            return lmin, outacc

        def row_body(g, carry):
            for r2 in range(2):
                ia = wid * NRW + g * 4 + r2 * 2
                ib = ia + 1
                xia = lane_bcast(xv, ia)
                yia = lane_bcast(yv, ia)
                zia = lane_bcast(zv, ia)
                sqia = lane_bcast(sqv, ia)
                xib = lane_bcast(xv, ib)
                yib = lane_bcast(yv, ib)
                zib = lane_bcast(zv, ib)
                sqib = lane_bcast(sqv, ib)

                def sc_body(sc, carry2):
                    lma, lmb = carry2
                    sacca = inf16
                    saccb = inf16
                    for u in range(16):
                        off = pl.multiple_of((sc * 16 + u) * 16, 16)
                        xj = xv[pl.ds(off, 16)]
                        yj = yv[pl.ds(off, 16)]
                        zj = zv[pl.ds(off, 16)]
                        sqj = sqv[pl.ds(off, 16)]
                        ta = xj * xia
                        ta = ta + yj * yia
                        ta = ta + zj * zia
                        sacca = jnp.minimum(sacca, (sqia + sqj) - 2.0 * ta)
                        tb = xj * xib
                        tb = tb + yj * yib
                        tb = tb + zj * zib
                        saccb = jnp.minimum(saccb, (sqib + sqj) - 2.0 * tb)
                    off2 = pl.multiple_of(sc * 16, 16)
                    smin2a[pl.ds(off2, 16)] = sacca
                    smin2b[pl.ds(off2, 16)] = saccb
                    return jnp.minimum(lma, sacca), jnp.minimum(lmb, saccb)

                lma, lmb = lax.fori_loop(0, NSC, sc_body, (inf16, inf16))

                def ext2_body(t, carry2):
                    lma, outa, lmb, outb = carry2
                    lma, outa = ext_one(t, lma, outa, smin2a, xia, yia, zia, sqia)
                    lmb, outb = ext_one(t, lmb, outb, smin2b, xib, yib, zib, sqib)
                    return lma, outa, lmb, outb

                zer = jnp.zeros((16,), jnp.int32)
                _, outa, _, outb = lax.fori_loop(0, K, ext2_body,
                                                 (lma, zer, lmb, zer))
                outv[pl.ds((r2 * 2) * K, K)] = jnp.minimum(outa, NP - 1)
                outv[pl.ds((r2 * 2 + 1) * K, K)] = jnp.minimum(outb, NP - 1)
            base_i = wid * NRW + g * 4
            pltpu.sync_copy(outv, idx_out.at[pl.ds(base_i * K, 4 * K)])
            return carry

        lax.fori_loop(0, NRW // 4, row_body, 0)

    return knn_kernel(xc, yc, zc)


def _attn_body(kvg_ref, q_ref, w1sc_ref, w1sh_ref, ww1_ref, bw1_ref,
               w2sc_ref, w2sh_ref, ww2_ref, bw2_ref, t16_ref, out_ref):
    kv = kvg_ref[...]
    kp = kv[:, :C]
    vp = kv[:, C:]
    q = q_ref[...]

    qb = jnp.broadcast_to(q[:, None, :], (RB, K, C)).reshape(RB * K, C)
    w = kp - qb
    w = jnp.maximum(w * w1sc_ref[...] + w1sh_ref[...], 0.0)
    w = jnp.dot(w, ww1_ref[...], preferred_element_type=jnp.float32) + bw1_ref[...]
    w = jnp.maximum(w * w2sc_ref[...] + w2sh_ref[...], 0.0)
    w = jnp.dot(w, ww2_ref[...], preferred_element_type=jnp.float32) + bw2_ref[...]

    w3 = w.reshape(RB, K, 16)
    m = jnp.max(w3, axis=1, keepdims=True)
    e = jnp.exp(w3 - m)
    sm = (e / jnp.sum(e, axis=1, keepdims=True)).reshape(RB * K, 16)
    wfull = jnp.dot(sm, t16_ref[...], preferred_element_type=jnp.float32)

    out_ref[...] = jnp.sum((vp * wfull).reshape(RB, K, C), axis=1)


def _sc_gather(kv_t, idxw):
    mesh = plsc.VectorSubcoreMesh(core_axis_name="c", subcore_axis_name="s")

    @functools.partial(
        pl.kernel, mesh=mesh,
        out_type=jax.ShapeDtypeStruct((NP * K, 2 * C), jnp.float32),
        scratch_types=[pltpu.VMEM((NCH, CH), jnp.int32),
                       pltpu.VMEM((CH, 2 * C), jnp.float32),
                       pltpu.SemaphoreType.DMA],
    )
    def gather_kernel(kv_hbm, idx_hbm, kvg_out, idx_v, rows_kv, sem):
        wid = lax.axis_index("s") * 2 + lax.axis_index("c")
        pltpu.sync_copy(idx_hbm.at[wid], idx_v)
        base = wid * PER_W

        def body(c, carry):
            pltpu.async_copy(kv_hbm.at[idx_v.at[c]], rows_kv, sem).wait()
            pltpu.sync_copy(rows_kv, kvg_out.at[pl.ds(base + c * CH, CH)])
            return carry

        lax.fori_loop(0, NCH, body, 0)

    return gather_kernel(kv_t, idxw)


def kernel(xyz, features, Wq, bq, Wk, bk, Wv, bv, Wp1, bp1, p_g, p_b, p_m, p_v,
           Wp2, bp2, w_g1, w_b1, w_m1, w_v1, Ww1, bw1, w_g2, w_b2, w_m2, w_v2,
           Ww2, bw2):
    f32 = jnp.float32

    xyz0 = xyz[0]
    xyz8 = jnp.zeros((NP, 8), f32)
    xyz8 = xyz8.at[:N, :3].set(xyz0)
    xyz8 = xyz8.at[N:, 0].set(FAR)
    x16 = jnp.concatenate([xyz8, jnp.zeros((NP, 8), f32)], axis=1)
    xc = jnp.full((NPC,), FAR, f32).at[:N].set(xyz0[:, 0])
    yc = jnp.zeros((NPC,), f32).at[:N].set(xyz0[:, 1])
    zc = jnp.zeros((NPC,), f32).at[:N].set(xyz0[:, 2])

    featT = jnp.pad(features[0].T, ((0, NP - N), (0, 0)))

    def bn_fold(g, b, m, v):
        sc = g / jnp.sqrt(v + EPS)
        return sc, b - m * sc

    psc, psh = bn_fold(p_g, p_b, p_m, p_v)
    w1sc, w1sh = bn_fold(w_g1, w_b1, w_m1, w_v1)
    w2sc, w2sh = bn_fold(w_g2, w_b2, w_m2, w_v2)

    wp1p = jnp.zeros((16, 16), f32).at[:3, :3].set(Wp1.T)
    bp1p = jnp.zeros((1, 16), f32).at[0, :3].set(bp1)
    pscp = jnp.ones((1, 16), f32).at[0, :3].set(psc)
    pshp = jnp.zeros((1, 16), f32).at[0, :3].set(psh)
    wp2p = jnp.zeros((16, C), f32).at[:3, :].set(Wp2.T)

    t16 = (lax.broadcasted_iota(jnp.int32, (16, C), 1) % 16 ==
           lax.broadcasted_iota(jnp.int32, (16, C), 0)).astype(f32)

    q_t, kv_t = pl.pallas_call(
        _prep_body,
        grid=(GRID,),
        in_specs=[pl.BlockSpec((RB, C), lambda i: (i, 0)),
                  pl.BlockSpec((RB, 16), lambda i: (i, 0))] +
                 [pl.BlockSpec((C, C), lambda i: (0, 0))] * 3 +
                 [pl.BlockSpec((1, C), lambda i: (0, 0))] * 3 +
                 [pl.BlockSpec((16, 16), lambda i: (0, 0)),
                  pl.BlockSpec((1, 16), lambda i: (0, 0)),
                  pl.BlockSpec((1, 16), lambda i: (0, 0)),
                  pl.BlockSpec((1, 16), lambda i: (0, 0)),
                  pl.BlockSpec((16, C), lambda i: (0, 0)),
                  pl.BlockSpec((1, C), lambda i: (0, 0))],
        out_specs=[pl.BlockSpec((RB, C), lambda i: (i, 0)),
                   pl.BlockSpec((RB, 2 * C), lambda i: (i, 0))],
        out_shape=[jax.ShapeDtypeStruct((NP, C), f32),
                   jax.ShapeDtypeStruct((NP, 2 * C), f32)],
    )(featT, x16, Wq.T, Wk.T, Wv.T, bq[None], bk[None], bv[None],
      wp1p, bp1p, pscp, pshp, wp2p, bp2[None])

    idx = _knn_sc(xc, yc, zc).reshape(NP, K)

    idxw = idx.reshape(NW, NCH, CH)
    kvg = _sc_gather(kv_t, idxw)

    wspec = lambda shape: pl.BlockSpec(shape, lambda i: (0, 0))
    out = pl.pallas_call(
        _attn_body,
        grid=(GRID,),
        in_specs=[pl.BlockSpec((RB * K, 2 * C), lambda i: (i, 0)),
                  pl.BlockSpec((RB, C), lambda i: (i, 0)),
                  wspec((1, C)), wspec((1, C)),
                  wspec((C, 16)), wspec((1, 16)), wspec((1, 16)), wspec((1, 16)),
                  wspec((16, 16)), wspec((1, 16)), wspec((16, C))],
        out_specs=pl.BlockSpec((RB, C), lambda i: (i, 0)),
        out_shape=jax.ShapeDtypeStruct((NP, C), f32),
    )(kvg, q_t,
      w1sc[None], w1sh[None],
      Ww1.T, bw1[None], w2sc[None], w2sh[None],
      Ww2.T, bw2[None], t16)

    return out[:N].T[None]


# speedup vs baseline: 4.2689x; 1.0746x over previous
import functools

import jax
import jax.numpy as jnp
from jax import lax
from jax.experimental import pallas as pl
from jax.experimental.pallas import tpu as pltpu
from jax.experimental.pallas import tpu_sc as plsc

N = 10000
K = 16
C = 128
NP = 10112
EPS = 1e-5

RB = 128
GRID = NP // RB

NW = 32
PER_W = NP * K // NW
CH = 64
NCH = PER_W // CH

FAR = 1.0e6


def _prep_body(f_ref, x16_ref, wq_ref, wk_ref, wv_ref, bq_ref, bk_ref, bv_ref,
               wp1_ref, bp1_ref, psc_ref, psh_ref, wp2_ref, bp2_ref,
               q_ref, kv_ref):
    f = f_ref[...]
    q = jnp.dot(f, wq_ref[...], preferred_element_type=jnp.float32) + bq_ref[...]
    k = jnp.dot(f, wk_ref[...], preferred_element_type=jnp.float32) + bk_ref[...]
    v = jnp.dot(f, wv_ref[...], preferred_element_type=jnp.float32) + bv_ref[...]
    x16 = x16_ref[...]
    p1 = jnp.dot(x16, wp1_ref[...], preferred_element_type=jnp.float32) + bp1_ref[...]
    p1 = jnp.maximum(p1 * psc_ref[...] + psh_ref[...], 0.0)
    p2 = jnp.dot(p1, wp2_ref[...], preferred_element_type=jnp.float32) + bp2_ref[...]
    q_ref[...] = q
    kv_ref[...] = jnp.concatenate([k + p2, v + p2], axis=1)


NPC = 10240
NSC = 40
NSCP = 48
NRW = NP // NW
FBIG = 3.0e38
IBIG = 2 ** 30


def _knn_sc(xc, yc, zc):
    mesh = plsc.VectorSubcoreMesh(core_axis_name="c", subcore_axis_name="s")

    @functools.partial(
        pl.kernel, mesh=mesh,
        out_type=jax.ShapeDtypeStruct((NP * K,), jnp.int32),
        scratch_types=[pltpu.VMEM((NPC,), jnp.float32),
                       pltpu.VMEM((NPC,), jnp.float32),
                       pltpu.VMEM((NPC,), jnp.float32),
                       pltpu.VMEM((NPC,), jnp.float32),
                       pltpu.VMEM((NSCP * 16,), jnp.float32),
                       pltpu.VMEM((NSCP * 16,), jnp.float32),
                       pltpu.VMEM((4 * K,), jnp.int32)],
        compiler_params=pltpu.CompilerParams(needs_layout_passes=False),
    )
    def knn_kernel(xc_hbm, yc_hbm, zc_hbm, idx_out, xv, yv, zv, sqv, smin2a,
                   smin2b, outv):
        iota = lax.iota(jnp.int32, 16)
        wid = lax.axis_index("s") * 2 + lax.axis_index("c")
        pltpu.sync_copy(xc_hbm, xv)
        pltpu.sync_copy(yc_hbm, yv)
        pltpu.sync_copy(zc_hbm, zv)
        inf16 = jnp.full((16,), FBIG, jnp.float32)
        for pads in range(NSC * 16, NSCP * 16, 16):
            smin2a[pl.ds(pads, 16)] = inf16
            smin2b[pl.ds(pads, 16)] = inf16

        def bf16_round(v):
            b = plsc.bitcast(v, jnp.int32)
            r = (b + 0x7FFF + ((b >> 16) & 1)) & ~0xFFFF
            return plsc.bitcast(r, jnp.float32)

        def pre_body(c, carry):
            off = pl.multiple_of(c * 16, 16)
            x = xv[pl.ds(off, 16)]
            y = yv[pl.ds(off, 16)]
            z = zv[pl.ds(off, 16)]
            sqv[pl.ds(off, 16)] = x * x + y * y + z * z
            xv[pl.ds(off, 16)] = bf16_round(x)
            yv[pl.ds(off, 16)] = bf16_round(y)
            zv[pl.ds(off, 16)] = bf16_round(z)
            return carry

        lax.fori_loop(0, NPC // 16, pre_body, 0)

        def lane_bcast(vec_ref, i):
            base = pl.multiple_of((i // 16) * 16, 16)
            chunk = vec_ref[pl.ds(base, 16)]
            s = jnp.min(jnp.where(iota == i % 16, chunk, FBIG))
            return jnp.full((16,), s, jnp.float32)

        def splat(x):
            return x if getattr(x, "ndim", 0) == 1 else jnp.full((16,), x)

        def ext_one(t, lmin, outacc, smin2, xi, yi, zi, sqi):
            m = jnp.min(lmin)
            mv = jnp.full((16,), m, jnp.float32)
            lstar = splat(plsc.all_reduce_ffs(lmin == mv))
            srows = []
            cands = []
            anys = []
            for kk in range(NSCP // 16):
                sidx = (kk * 16 + iota) * 16 + lstar
                srow = plsc.load_gather(smin2, [sidx])
                srows.append(srow)
                eq = srow == mv
                cands.append(splat(plsc.all_reduce_ffs(eq)) + kk * 16)
                anys.append(splat(plsc.all_reduce_population_count(eq)))
            sstar = jnp.where(anys[0] > 0, cands[0],
                              jnp.where(anys[1] > 0, cands[1], cands[2]))
            jv = (sstar * 16 + iota) * 16 + lstar
            t1 = plsc.load_gather(xv, [jv]) * xi
            t1 = t1 + plsc.load_gather(yv, [jv]) * yi
            t1 = t1 + plsc.load_gather(zv, [jv]) * zi
            d2v = (sqi + plsc.load_gather(sqv, [jv])) - 2.0 * t1
            cstar = sstar * 16 + splat(plsc.all_reduce_ffs(d2v == mv))
            jstar = cstar * 16 + lstar
            outacc = jnp.where(iota == t, jstar, outacc)
            cmin = jnp.min(jnp.where(d2v <= mv, FBIG, d2v))
            cminv = jnp.full((16,), cmin, jnp.float32)
            plsc.store_scatter(smin2, [sstar * 16 + lstar], cminv,
                               mask=iota == 0)
            lnew = inf16
            sstarv = sstar
            for kk in range(NSCP // 16):
                srow = jnp.where(kk * 16 + iota == sstarv, cminv, srows[kk])
                lnew = jnp.minimum(lnew, srow)
            lmin = jnp.where(iota == lstar,
                             jnp.full((16,), jnp.min(lnew), jnp.float32), lmin)
            return lmin, outacc

        def row_body(g, carry):
            for r2 in range(2):
                ia = wid * NRW + g * 4 + r2 * 2
                ib = ia + 1
                xia = lane_bcast(xv, ia)
                yia = lane_bcast(yv, ia)
                zia = lane_bcast(zv, ia)
                sqia = lane_bcast(sqv, ia)
                xib = lane_bcast(xv, ib)
                yib = lane_bcast(yv, ib)
                zib = lane_bcast(zv, ib)
                sqib = lane_bcast(sqv, ib)

                def sc_body(sc, carry2):
                    lma, lmb = carry2
                    sacca = inf16
                    saccb = inf16
                    for u in range(16):
                        off = pl.multiple_of((sc * 16 + u) * 16, 16)
                        xj = xv[pl.ds(off, 16)]
                        yj = yv[pl.ds(off, 16)]
                        zj = zv[pl.ds(off, 16)]
                        sqj = sqv[pl.ds(off, 16)]
                        ta = xj * xia
                        ta = ta + yj * yia
                        ta = ta + zj * zia
                        sacca = jnp.minimum(sacca, (sqia + sqj) - 2.0 * ta)
                        tb = xj * xib
                        tb = tb + yj * yib
                        tb = tb + zj * zib
                        saccb = jnp.minimum(saccb, (sqib + sqj) - 2.0 * tb)
                    off2 = pl.multiple_of(sc * 16, 16)
                    smin2a[pl.ds(off2, 16)] = sacca
                    smin2b[pl.ds(off2, 16)] = saccb
                    return jnp.minimum(lma, sacca), jnp.minimum(lmb, saccb)

                lma, lmb = lax.fori_loop(0, NSC, sc_body, (inf16, inf16))

                def ext2_body(t, carry2):
                    lma, outa, lmb, outb = carry2
                    lma, outa = ext_one(t, lma, outa, smin2a, xia, yia, zia, sqia)
                    lmb, outb = ext_one(t, lmb, outb, smin2b, xib, yib, zib, sqib)
                    return lma, outa, lmb, outb

                zer = jnp.zeros((16,), jnp.int32)
                _, outa, _, outb = lax.fori_loop(0, K, ext2_body,
                                                 (lma, zer, lmb, zer))
                outv[pl.ds((r2 * 2) * K, K)] = jnp.minimum(outa, NP - 1)
                outv[pl.ds((r2 * 2 + 1) * K, K)] = jnp.minimum(outb, NP - 1)
            base_i = wid * NRW + g * 4
            pltpu.sync_copy(outv, idx_out.at[pl.ds(base_i * K, 4 * K)])
            return carry

        lax.fori_loop(0, NRW // 4, row_body, 0)

    return knn_kernel(xc, yc, zc)


def _attn_body(kvg_ref, q_ref, w1sc_ref, w1sh_ref, ww1_ref, bw1_ref,
               w2sc_ref, w2sh_ref, ww2_ref, bw2_ref, t16_ref, out_ref):
    kv = kvg_ref[...]
    kp = kv[:, :C]
    vp = kv[:, C:]
    q = q_ref[...]

    qb = jnp.broadcast_to(q[:, None, :], (RB, K, C)).reshape(RB * K, C)
    w = kp - qb
    w = jnp.maximum(w * w1sc_ref[...] + w1sh_ref[...], 0.0)
    w = jnp.dot(w, ww1_ref[...], preferred_element_type=jnp.float32) + bw1_ref[...]
    w = jnp.maximum(w * w2sc_ref[...] + w2sh_ref[...], 0.0)
    w = jnp.dot(w, ww2_ref[...], preferred_element_type=jnp.float32) + bw2_ref[...]

    w3 = w.reshape(RB, K, 16)
    m = jnp.max(w3, axis=1, keepdims=True)
    e = jnp.exp(w3 - m)
    sm = (e / jnp.sum(e, axis=1, keepdims=True)).reshape(RB * K, 16)
    wfull = jnp.dot(sm, t16_ref[...], preferred_element_type=jnp.float32)

    out_ref[...] = jnp.sum((vp * wfull).reshape(RB, K, C), axis=1)


def _sc_gather(kv_t, idxw):
    mesh = plsc.VectorSubcoreMesh(core_axis_name="c", subcore_axis_name="s")

    @functools.partial(
        pl.kernel, mesh=mesh,
        out_type=jax.ShapeDtypeStruct((NP * K, 2 * C), jnp.float32),
        scratch_types=[pltpu.VMEM((NCH, CH), jnp.int32),
                       pltpu.VMEM((CH, 2 * C), jnp.float32),
                       pltpu.SemaphoreType.DMA],
    )
    def gather_kernel(kv_hbm, idx_hbm, kvg_out, idx_v, rows_kv, sem):
        wid = lax.axis_index("s") * 2 + lax.axis_index("c")
        pltpu.sync_copy(idx_hbm.at[wid], idx_v)
        base = wid * PER_W

        def body(c, carry):
            pltpu.async_copy(kv_hbm.at[idx_v.at[c]], rows_kv, sem).wait()
            pltpu.sync_copy(rows_kv, kvg_out.at[pl.ds(base + c * CH, CH)])
            return carry

        lax.fori_loop(0, NCH, body, 0)

    return gather_kernel(kv_t, idxw)


def kernel(xyz, features, Wq, bq, Wk, bk, Wv, bv, Wp1, bp1, p_g, p_b, p_m, p_v,
           Wp2, bp2, w_g1, w_b1, w_m1, w_v1, Ww1, bw1, w_g2, w_b2, w_m2, w_v2,
           Ww2, bw2):
    f32 = jnp.float32

    xyz0 = xyz[0]
    xyz8 = jnp.zeros((NP, 8), f32)
    xyz8 = xyz8.at[:N, :3].set(xyz0)
    xyz8 = xyz8.at[N:, 0].set(FAR)
    x16 = jnp.concatenate([xyz8, jnp.zeros((NP, 8), f32)], axis=1)
    xc = jnp.full((NPC,), FAR, f32).at[:N].set(xyz0[:, 0])
    yc = jnp.zeros((NPC,), f32).at[:N].set(xyz0[:, 1])
    zc = jnp.zeros((NPC,), f32).at[:N].set(xyz0[:, 2])

    featT = jnp.pad(features[0].T, ((0, NP - N), (0, 0)))

    def bn_fold(g, b, m, v):
        sc = g / jnp.sqrt(v + EPS)
        return sc, b - m * sc

    psc, psh = bn_fold(p_g, p_b, p_m, p_v)
    w1sc, w1sh = bn_fold(w_g1, w_b1, w_m1, w_v1)
    w2sc, w2sh = bn_fold(w_g2, w_b2, w_m2, w_v2)

    wp1p = jnp.zeros((16, 16), f32).at[:3, :3].set(Wp1.T)
    bp1p = jnp.zeros((1, 16), f32).at[0, :3].set(bp1)
    pscp = jnp.ones((1, 16), f32).at[0, :3].set(psc)
    pshp = jnp.zeros((1, 16), f32).at[0, :3].set(psh)
    wp2p = jnp.zeros((16, C), f32).at[:3, :].set(Wp2.T)

    t16 = (lax.broadcasted_iota(jnp.int32, (16, C), 1) % 16 ==
           lax.broadcasted_iota(jnp.int32, (16, C), 0)).astype(f32)

    q_t, kv_t = pl.pallas_call(
        _prep_body,
        grid=(GRID,),
        in_specs=[pl.BlockSpec((RB, C), lambda i: (i, 0)),
                  pl.BlockSpec((RB, 16), lambda i: (i, 0))] +
                 [pl.BlockSpec((C, C), lambda i: (0, 0))] * 3 +
                 [pl.BlockSpec((1, C), lambda i: (0, 0))] * 3 +
                 [pl.BlockSpec((16, 16), lambda i: (0, 0)),
                  pl.BlockSpec((1, 16), lambda i: (0, 0)),
                  pl.BlockSpec((1, 16), lambda i: (0, 0)),
                  pl.BlockSpec((1, 16), lambda i: (0, 0)),
                  pl.BlockSpec((16, C), lambda i: (0, 0)),
                  pl.BlockSpec((1, C), lambda i: (0, 0))],
        out_specs=[pl.BlockSpec((RB, C), lambda i: (i, 0)),
                   pl.BlockSpec((RB, 2 * C), lambda i: (i, 0))],
        out_shape=[jax.ShapeDtypeStruct((NP, C), f32),
                   jax.ShapeDtypeStruct((NP, 2 * C), f32)],
    )(featT, x16, Wq.T, Wk.T, Wv.T, bq[None], bk[None], bv[None],
      wp1p, bp1p, pscp, pshp, wp2p, bp2[None])

    idx = _knn_sc(xc, yc, zc).reshape(NP, K)

    idxw = idx.reshape(NW, NCH, CH)
    kvg = _sc_gather(kv_t, idxw)

    wspec = lambda shape: pl.BlockSpec(shape, lambda i: (0, 0))
    out = pl.pallas_call(
        _attn_body,
        grid=(GRID,),
        in_specs=[pl.BlockSpec((RB * K, 2 * C), lambda i: (i, 0)),
                  pl.BlockSpec((RB, C), lambda i: (i, 0)),
                  wspec((1, C)), wspec((1, C)),
                  wspec((C, 16)), wspec((1, 16)), wspec((1, 16)), wspec((1, 16)),
                  wspec((16, 16)), wspec((1, 16)), wspec((16, C))],
        out_specs=pl.BlockSpec((RB, C), lambda i: (i, 0)),
        out_shape=jax.ShapeDtypeStruct((NP, C), f32),
    )(kvg, q_t,
      w1sc[None], w1sh[None],
      Ww1.T, bw1[None], w2sc[None], w2sh[None],
      Ww2.T, bw2[None], t16)

    return out[:N].T[None]
